# baseline (speedup 1.0000x reference)
import functools

import jax
import jax.numpy as jnp
from jax import lax
from jax.experimental import pallas as pl
from jax.experimental.pallas import tpu as pltpu
from jax.experimental.pallas import tpu_sc as plsc

N = 100000
L = 200000
K = 4

NC = 2
NS = 16
NW = NC * NS

CA = 6256
CL = 12544
CLQ = CL // 4
CN = 3136
CNQ = CN // 2

_mesh = plsc.VectorSubcoreMesh(core_axis_name="c", subcore_axis_name="s")
_params = pltpu.CompilerParams(needs_layout_passes=False)


@functools.partial(
    pl.kernel,
    out_type=jax.ShapeDtypeStruct((N,), jnp.float32),
    mesh=_mesh,
    compiler_params=_params,
    scratch_types=[
        pltpu.VMEM_SHARED((N,), jnp.float32),
        pltpu.VMEM_SHARED((L,), jnp.float32),
        pltpu.VMEM((CL,), jnp.int32),
        pltpu.VMEM((CL,), jnp.int32),
        pltpu.VMEM((CL,), jnp.float32),
        pltpu.VMEM((CL,), jnp.float32),
        pltpu.VMEM((CL,), jnp.float32),
        pltpu.VMEM((CL,), jnp.float32),
        [pltpu.VMEM((CN,), jnp.int32) for _ in range(K)],
        [pltpu.VMEM((CN,), jnp.float32) for _ in range(K)],
        pltpu.VMEM((CN,), jnp.float32),
        [pltpu.SemaphoreType.DMA for _ in range(13)],
    ],
)
def _slope_fused(head_hbm, tail_hbm, len_hbm, array_hbm, linksT_hbm, out_hbm,
                 arr_s, grad_s,
                 head_v, tail_v, len_v, hval_v, tval_v, grad_v,
                 links_v, g_v, out_v,
                 sems):
    cid = lax.axis_index("c")
    sid = lax.axis_index("s")
    wid = sid * NC + cid

    lbase = jnp.minimum(sid * CL, L - CL)
    cp_h = pltpu.async_copy(head_hbm.at[pl.ds(lbase, CL)], head_v, sems[8])
    cp_t = pltpu.async_copy(tail_hbm.at[pl.ds(lbase, CL)], tail_v, sems[9])
    cp_l = pltpu.async_copy(len_hbm.at[pl.ds(lbase, CL)], len_v, sems[10])

    nbase = jnp.minimum(wid * CN, N - CN)
    idx_cps = [
        pltpu.async_copy(linksT_hbm.at[pl.ds(j * N + nbase, CN)], links_v[j],
                         sems[4 + j])
        for j in range(K)
    ]

    abase = jnp.minimum(sid * CA, N - CA)
    pltpu.sync_copy(array_hbm.at[pl.ds(abase, CA)], grad_v.at[pl.ds(0, CA)])
    cp_a = pltpu.async_copy(grad_v.at[pl.ds(0, CA)],
                            arr_s.at[pl.ds(abase, CA)], sems[12])

    cp_a.wait()
    plsc.subcore_barrier()
    cp_h.wait()
    cp_t.wait()
    cp_l.wait()

    g_h = [
        pltpu.async_copy(arr_s.at[head_v.at[pl.ds(q * CLQ, CLQ)]],
                         hval_v.at[pl.ds(q * CLQ, CLQ)], sems[q])
        for q in range(4)
    ]
    g_t = [
        pltpu.async_copy(arr_s.at[tail_v.at[pl.ds(q * CLQ, CLQ)]],
                         tval_v.at[pl.ds(q * CLQ, CLQ)], sems[8 + q])
        for q in range(4)
    ]

    def body_a(i, carry):
        ds = pl.ds(i * 16, 16)
        grad_v[ds] = (hval_v[ds] - tval_v[ds]) / len_v[ds]
        return carry

    w_cps = []
    for q in range(4):
        g_h[q].wait()
        g_t[q].wait()
        lax.fori_loop(q * (CLQ // 16), (q + 1) * (CLQ // 16), body_a, 0)
        w_cps.append(pltpu.async_copy(grad_v.at[pl.ds(q * CLQ, CLQ)],
                                      grad_s.at[pl.ds(lbase + q * CLQ, CLQ)],
                                      sems[q]))
    for cp in w_cps:
        cp.wait()
    plsc.subcore_barrier()

    for cp in idx_cps:
        cp.wait()
    gb = [
        [
            pltpu.async_copy(grad_s.at[links_v[j].at[pl.ds(q * CNQ, CNQ)]],
                             g_v[j].at[pl.ds(q * CNQ, CNQ)], sems[4 * q + j])
            for j in range(K)
        ]
        for q in range(2)
    ]

    def body_b(i, carry):
        ds = pl.ds(i * 16, 16)
        out_v[ds] = ((g_v[0][ds] + g_v[1][ds]) + (g_v[2][ds] + g_v[3][ds])) * 0.25
        return carry

    w_out = []
    for q in range(2):
        for cp in gb[q]:
            cp.wait()
        lax.fori_loop(q * (CNQ // 16), (q + 1) * (CNQ // 16), body_b, 0)
        w_out.append(pltpu.async_copy(out_v.at[pl.ds(q * CNQ, CNQ)],
                                      out_hbm.at[pl.ds(nbase + q * CNQ, CNQ)],
                                      sems[8 + q]))
    for cp in w_out:
        cp.wait()


def kernel(array, length_of_link, node_at_link_head, node_at_link_tail, links_at_node):
    linksT = links_at_node.T.reshape(-1)
    return _slope_fused(node_at_link_head, node_at_link_tail, length_of_link,
                        array, linksT)


# speedup vs baseline: 1.0282x; 1.0282x over previous
import functools

import jax
import jax.numpy as jnp
from jax import lax
from jax.experimental import pallas as pl
from jax.experimental.pallas import tpu as pltpu
from jax.experimental.pallas import tpu_sc as plsc

N = 100000
L = 200000
K = 4

NC = 2
NS = 16
NW = NC * NS

CA = 6256
CL = 12544
CLQ = CL // 4
CN = 3136
CNQ = CN // 2

_mesh = plsc.VectorSubcoreMesh(core_axis_name="c", subcore_axis_name="s")
_params = pltpu.CompilerParams(needs_layout_passes=False)


@functools.partial(
    pl.kernel,
    out_type=(jax.ShapeDtypeStruct((N,), jnp.float32),
              jax.ShapeDtypeStruct((L,), jnp.float32)),
    mesh=_mesh,
    compiler_params=_params,
    scratch_types=[
        pltpu.VMEM_SHARED((N,), jnp.float32),
        pltpu.VMEM_SHARED((L,), jnp.float32),
        pltpu.VMEM((CL,), jnp.int32),
        pltpu.VMEM((CL,), jnp.int32),
        pltpu.VMEM((CL,), jnp.float32),
        pltpu.VMEM((CL,), jnp.float32),
        pltpu.VMEM((CL,), jnp.float32),
        pltpu.VMEM((CL,), jnp.float32),
        [pltpu.VMEM((CN,), jnp.int32) for _ in range(K)],
        [pltpu.VMEM((CN,), jnp.float32) for _ in range(K)],
        pltpu.VMEM((CN,), jnp.float32),
        [pltpu.SemaphoreType.DMA for _ in range(16)],
    ],
)
def _slope_fused(head_hbm, tail_hbm, len_hbm, array_hbm, linksT_hbm,
                 out_hbm, grad_hbm,
                 arr_s, grad_s,
                 head_v, tail_v, len_v, hval_v, tval_v, grad_v,
                 links_v, g_v, out_v,
                 sems):
    cid = lax.axis_index("c")
    sid = lax.axis_index("s")
    wid = sid * NC + cid

    lbase = jnp.minimum(sid * CL, L - CL)
    cp_h = pltpu.async_copy(head_hbm.at[pl.ds(lbase, CL)], head_v, sems[8])
    cp_t = pltpu.async_copy(tail_hbm.at[pl.ds(lbase, CL)], tail_v, sems[9])
    cp_l = pltpu.async_copy(len_hbm.at[pl.ds(lbase, CL)], len_v, sems[10])

    nbase = jnp.minimum(wid * CN, N - CN)
    idx_cps = [
        pltpu.async_copy(linksT_hbm.at[pl.ds(j * N + nbase, CN)], links_v[j],
                         sems[4 + j])
        for j in range(K)
    ]

    abase = jnp.minimum(sid * CA, N - CA)
    pltpu.sync_copy(array_hbm.at[pl.ds(abase, CA)], grad_v.at[pl.ds(0, CA)])
    cp_a = pltpu.async_copy(grad_v.at[pl.ds(0, CA)],
                            arr_s.at[pl.ds(abase, CA)], sems[12])

    cp_a.wait()
    plsc.subcore_barrier()
    cp_h.wait()
    cp_t.wait()
    cp_l.wait()

    g_h = [
        pltpu.async_copy(arr_s.at[head_v.at[pl.ds(q * CLQ, CLQ)]],
                         hval_v.at[pl.ds(q * CLQ, CLQ)], sems[q])
        for q in range(4)
    ]
    g_t = [
        pltpu.async_copy(
            (array_hbm if q == 3 else arr_s).at[tail_v.at[pl.ds(q * CLQ, CLQ)]],
            tval_v.at[pl.ds(q * CLQ, CLQ)], sems[8 + q])
        for q in range(4)
    ]

    def body_a(i, carry):
        ds = pl.ds(i * 16, 16)
        grad_v[ds] = (hval_v[ds] - tval_v[ds]) / len_v[ds]
        return carry

    w_cps = []
    for q in range(4):
        g_h[q].wait()
        g_t[q].wait()
        lax.fori_loop(q * (CLQ // 16), (q + 1) * (CLQ // 16), body_a, 0)
        w_cps.append(pltpu.async_copy(grad_v.at[pl.ds(q * CLQ, CLQ)],
                                      grad_s.at[pl.ds(lbase + q * CLQ, CLQ)],
                                      sems[q]))
        w_cps.append(pltpu.async_copy(grad_v.at[pl.ds(q * CLQ, CLQ)],
                                      grad_hbm.at[pl.ds(lbase + q * CLQ, CLQ)],
                                      sems[12 + q]))
    for cp in w_cps:
        cp.wait()
    plsc.subcore_barrier()

    for cp in idx_cps:
        cp.wait()
    gb = [
        [
            pltpu.async_copy(
                (grad_hbm if (q, j) == (1, 3) else grad_s)
                .at[links_v[j].at[pl.ds(q * CNQ, CNQ)]],
                g_v[j].at[pl.ds(q * CNQ, CNQ)], sems[4 * q + j])
            for j in range(K)
        ]
        for q in range(2)
    ]

    def body_b(i, carry):
        ds = pl.ds(i * 16, 16)
        out_v[ds] = ((g_v[0][ds] + g_v[1][ds]) + (g_v[2][ds] + g_v[3][ds])) * 0.25
        return carry

    w_out = []
    for q in range(2):
        for cp in gb[q]:
            cp.wait()
        lax.fori_loop(q * (CNQ // 16), (q + 1) * (CNQ // 16), body_b, 0)
        w_out.append(pltpu.async_copy(out_v.at[pl.ds(q * CNQ, CNQ)],
                                      out_hbm.at[pl.ds(nbase + q * CNQ, CNQ)],
                                      sems[8 + q]))
    for cp in w_out:
        cp.wait()


def kernel(array, length_of_link, node_at_link_head, node_at_link_tail, links_at_node):
    linksT = links_at_node.T.reshape(-1)
    slope, _ = _slope_fused(node_at_link_head, node_at_link_tail,
                            length_of_link, array, linksT)
    return slope
